# Initial kernel scaffold; baseline (speedup 1.0000x reference)
import jax
import jax.numpy as jnp
from jax.experimental import pallas as pl


def kernel(input_words, output_words, noise_words, in_embed_weight, out_embed_weight):
    raise NotImplementedError("write your pallas kernel here")


# speedup vs baseline: 1.7403x; 1.7403x over previous
import functools

import jax
import jax.numpy as jnp
from jax import lax
from jax.experimental import pallas as pl
from jax.experimental.pallas import tpu as pltpu
from jax.experimental.pallas import tpu_sc as plsc

V = 1000000
D = 64
B = 16384
NS = 5

NC = 2
NSC = 16
NW = NC * NSC
EPW = B // NW
C = 128
NCHUNK = EPW // C
NR = C * NS

_mesh = plsc.VectorSubcoreMesh(
    core_axis_name="c", subcore_axis_name="s", num_cores=NC, num_subcores=NSC
)


@functools.partial(
    pl.kernel,
    out_type=jax.ShapeDtypeStruct((NW, 1 + NS, EPW), jnp.float32),
    mesh=_mesh,
    compiler_params=pltpu.CompilerParams(
        needs_layout_passes=False, use_tc_tiling_on_sc=False),
    scratch_types=[
        pltpu.VMEM((C,), jnp.int32),
        pltpu.VMEM((C,), jnp.int32),
        pltpu.VMEM((NR,), jnp.int32),
        pltpu.VMEM((C, D), jnp.float32),
        pltpu.VMEM((C, D), jnp.float32),
        pltpu.VMEM((NR, D), jnp.float32),
        pltpu.VMEM((1 + NS, C), jnp.float32),
        pltpu.SemaphoreType.DMA,
    ],
)
def _sc_dots(in_w, out_w, noise_w, in_tbl, out_tbl, dots_hbm,
             iidx, oidx, nidx, irows, orows, nrows, dots_v, sem):
    wid = lax.axis_index("s") * NC + lax.axis_index("c")
    lane = lax.iota(jnp.int32, 16)

    def chunk_body(c_i, carry):
        base = wid * EPW + c_i * C

        pltpu.sync_copy(in_w.at[pl.ds(base, C)], iidx)
        pltpu.sync_copy(out_w.at[pl.ds(base, C)], oidx)
        pltpu.sync_copy(noise_w.at[pl.ds(base * NS, NR)], nidx)

        cps = [
            pltpu.async_copy(in_tbl.at[iidx], irows, sem),
            pltpu.async_copy(out_tbl.at[oidx], orows, sem),
        ]
        for n in range(NS):
            cps.append(
                pltpu.async_copy(out_tbl.at[nidx.at[pl.ds(n * C, C)]],
                                 nrows.at[pl.ds(n * C, C)], sem))
        for cp in cps:
            cp.wait()

        def group_body(g, carry2):
            accs = [jnp.zeros((16,), jnp.float32) for _ in range(1 + NS)]
            for l in range(16):
                e = g * 16 + l
                iv = [irows[e, pl.ds(k * 16, 16)] for k in range(D // 16)]
                ov = [orows[e, pl.ds(k * 16, 16)] for k in range(D // 16)]
                p = iv[0] * ov[0]
                for k in range(1, D // 16):
                    p = p + iv[k] * ov[k]
                msk = lane == l
                accs[0] = jnp.where(msk, jnp.sum(p), accs[0])
                for n in range(NS):
                    r = e * NS + n
                    q = iv[0] * nrows[r, pl.ds(0, 16)]
                    for k in range(1, D // 16):
                        q = q + iv[k] * nrows[r, pl.ds(k * 16, 16)]
                    accs[1 + n] = jnp.where(msk, jnp.sum(q), accs[1 + n])
            for j in range(1 + NS):
                dots_v[j, pl.ds(g * 16, 16)] = accs[j]
            return carry2

        lax.fori_loop(0, C // 16, group_body, 0)

        pltpu.sync_copy(dots_v, dots_hbm.at[wid, :, pl.ds(c_i * C, C)])
        return carry

    lax.fori_loop(0, NCHUNK, chunk_body, 0)


def _tc_loss_kernel(dots_ref, out_ref):
    x = dots_ref[...]
    row = lax.broadcasted_iota(jnp.int32, x.shape, 1)
    t = jnp.where(row == 0, x, -x)
    terms = jnp.log(1.0 / (1.0 + jnp.exp(-t)))
    out_ref[0, 0] = -jnp.sum(terms) / B


_tc_loss = pl.pallas_call(
    _tc_loss_kernel,
    out_shape=jax.ShapeDtypeStruct((1, 1), jnp.float32),
    out_specs=pl.BlockSpec(memory_space=pltpu.SMEM),
)


def kernel(input_words, output_words, noise_words, in_embed_weight, out_embed_weight):
    dots = _sc_dots(input_words, output_words, noise_words,
                    in_embed_weight, out_embed_weight)
    return _tc_loss(dots)[0, 0]
